# baseline (speedup 1.0000x reference)
import functools

import numpy as np
import jax
import jax.numpy as jnp
from jax import lax
from jax.experimental import pallas as pl
from jax.experimental.pallas import tpu as pltpu
from jax.experimental.pallas import tpu_sc as plsc

_NUM_DET = 100
_NUM_CLASSES = 80
_ROW = 85
_LANES = 16
_WORKERS = 7
_PAD_DET = _WORKERS * _LANES
_OUT_COLS = 7
_SLOTS = 3

_BATCHES = np.array(
    [0, 0, 0, 0, 0, 0, 0, 0, 0, 0, 0, 0, 0, 1, 1, 1, 1, 1, 1, 1,
     1, 1, 1, 1, 1, 1, 1, 1, 2, 2, 2, 2, 2, 2, 2, 2, 2, 2, 2, 3,
     3, 3, 3, 3, 3, 3, 3, 3, 3, 3, 3, 3, 3, 3, 3, 4, 4, 4, 4, 4,
     4, 4, 4, 4, 4, 4, 4, 4, 4, 4, 4, 5, 5, 5, 5, 5, 5, 5, 5, 5,
     5, 6, 6, 6, 6, 6, 6, 6, 6, 7, 7, 7, 7, 7, 7, 7, 7, 7, 7, 7],
    dtype=np.int32)


def _worker_plan():
    plan = []
    for w in range(_WORKERS):
        slots, bounds = [], []
        for j in range(_LANES):
            i = 16 * w + j
            b = int(_BATCHES[min(i, _NUM_DET - 1)])
            if not slots:
                slots.append(b)
            elif b != slots[-1]:
                slots.append(b)
                bounds.append(j)
        while len(bounds) < _SLOTS - 1:
            bounds.append(_LANES)
        plan.append((slots, bounds))
    return plan


_PLAN = _worker_plan()


def _sc_body(x_hbm, out_hbm, stage_v, lidx_v, bf_v, out_v):
    wid = lax.axis_index("s") * 2 + lax.axis_index("c")

    @pl.when(wid < _WORKERS)
    def _():
        lanes = lax.iota(jnp.int32, _LANES)

        for k, (slots, bounds) in enumerate(_PLAN):
            @pl.when(wid == k)
            def _(k=k, slots=slots, bounds=bounds):
                for s, b in enumerate(slots):
                    pltpu.sync_copy(
                        x_hbm.at[b, pl.ds(16 * k, _LANES), :],
                        stage_v.at[pl.ds(s * _LANES, _LANES), :])
                ge1 = (lanes >= bounds[0]).astype(jnp.int32)
                ge2 = (lanes >= bounds[1]).astype(jnp.int32)
                lidx_v[...] = (ge1 + ge2) * _LANES + lanes
                full = slots + [slots[-1]] * (_SLOTS - len(slots))
                bf_v[...] = (full[0] + ge1 * (full[1] - full[0])
                             + ge2 * (full[2] - full[1])
                             ).astype(jnp.float32)

        lidx = lidx_v[...]

        def col(c):
            return plsc.load_gather(
                stage_v, [lidx, jnp.full((_LANES,), c, jnp.int32)])

        cx, cy, bw, bh = col(0), col(1), col(2), col(3)
        conf = col(4)
        half = jnp.float32(0.5)
        x1 = cx - half * bw
        y1 = cy - half * bh
        x2 = cx + half * bw
        y2 = cy + half * bh

        best = col(5) * conf
        best_c = jnp.zeros((_LANES,), jnp.int32)
        for c in range(1, _NUM_CLASSES):
            s = col(5 + c) * conf
            gt = s > best
            best = jnp.where(gt, s, best)
            best_c = jnp.where(gt, jnp.full((_LANES,), c, jnp.int32), best_c)

        base = lax.iota(jnp.int32, _LANES) * _OUT_COLS
        outs = (bf_v[...], x1, y1, x2, y2,
                best_c.astype(jnp.float32), best)
        for c, v in enumerate(outs):
            plsc.store_scatter(out_v, [base + c], v)
        pltpu.sync_copy(out_v, out_hbm.at[wid, 0])


@functools.lru_cache(maxsize=None)
def _build_sc_call():
    mesh = plsc.VectorSubcoreMesh(core_axis_name="c", subcore_axis_name="s")
    return pl.kernel(
        _sc_body,
        out_type=jax.ShapeDtypeStruct((_WORKERS, 1, _LANES * _OUT_COLS),
                                      jnp.float32),
        mesh=mesh,
        scratch_types=[
            pltpu.VMEM((_SLOTS * _LANES, _ROW), jnp.float32),
            pltpu.VMEM((_LANES,), jnp.int32),
            pltpu.VMEM((_LANES,), jnp.float32),
            pltpu.VMEM((_LANES * _OUT_COLS,), jnp.float32),
        ],
        compiler_params=pltpu.CompilerParams(needs_layout_passes=False),
    )


def kernel(x):
    xw = lax.slice(x, (0, 100, 0), (8, 100 + _PAD_DET, _ROW))
    out = _build_sc_call()(xw)
    return out.reshape(_PAD_DET, _OUT_COLS)[:_NUM_DET]
